# baseline (speedup 1.0000x reference)
import functools

import jax
import jax.numpy as jnp
from jax import lax
from jax.experimental import pallas as pl
from jax.experimental.pallas import tpu as pltpu
from jax.experimental.pallas import tpu_sc as plsc

N = 320000
DIM = 128
NSEG = 10000
NSEG_PAD = 10240
NC = 2
NS = 16
NW = NC * NS
ROWS_PER_TILE = N // NW
CHUNK = 64
NCH = ROWS_PER_TILE // CHUNK
TAIL = ROWS_PER_TILE - NCH * CHUNK
RB = 3
PB = 2
NG = DIM // 16
WB = 632
WB_LAST = NSEG - 15 * WB
CNT_PER_TILE = NSEG_PAD // NS


def _phase_a(node_input, labels):
    mesh = plsc.VectorSubcoreMesh(core_axis_name="c", subcore_axis_name="s")

    @functools.partial(
        pl.kernel,
        out_type=[
            jax.ShapeDtypeStruct((NC, NSEG, DIM), jnp.float32),
            jax.ShapeDtypeStruct((NC, NSEG_PAD), jnp.float32),
        ],
        mesh=mesh,
        compiler_params=pltpu.CompilerParams(needs_layout_passes=False),
        scratch_types=[
            pltpu.VMEM_SHARED((NSEG, DIM), jnp.float32),
            pltpu.VMEM_SHARED((NSEG_PAD,), jnp.float32),
            pltpu.VMEM((RB * CHUNK, DIM), jnp.float32),
            pltpu.VMEM((RB, 80), jnp.int32),
            pltpu.VMEM((PB, CHUNK, DIM), jnp.float32),
            pltpu.VMEM((PB, CHUNK), jnp.int32),
            pltpu.VMEM((PB, CHUNK), jnp.int32),
            pltpu.VMEM((CHUNK,), jnp.float32),
            pltpu.VMEM((CHUNK,), jnp.float32),
            pltpu.VMEM((CHUNK,), jnp.float32),
            pltpu.VMEM((CNT_PER_TILE,), jnp.float32),
            pltpu.VMEM((32,), jnp.int32),
            pltpu.VMEM((16,), jnp.int32),
            pltpu.VMEM((16,), jnp.int32),
            pltpu.SemaphoreType.DMA((RB,)),
            pltpu.SemaphoreType.DMA((PB,)),
        ],
    )
    def k(node_h, lab_h, sums_h, cnt_h, acc, cacc, rbuf, lbuf, pbuf, mlab,
          nlab, sbuf, ccs, nccs, zcnt, tlab, mtail, ntail, gsem, ssem):
        cid = lax.axis_index("c")
        sid = lax.axis_index("s")
        row0 = cid * (N // NC) + sid * ROWS_PER_TILE

        zeros16 = jnp.zeros((16,), jnp.float32)
        iota16 = lax.iota(jnp.int32, 16)
        lane0 = iota16 != 0
        lane15 = iota16 == 15
        neg1 = jnp.full((16,), -1, jnp.int32)

        def fill_zrow(i, carry):
            for g in range(NG):
                rbuf[i, pl.ds(g * 16, 16)] = zeros16
            return carry

        lax.fori_loop(0, RB * CHUNK, fill_zrow, 0)

        def fill_zcnt(i, carry):
            zcnt[pl.ds(i * 16, 16)] = zeros16
            return carry

        lax.fori_loop(0, CNT_PER_TILE // 16, fill_zcnt, 0)

        for j in range(CHUNK // 16):
            pos = (iota16 + (j * 16 + 1)).astype(jnp.float32)
            ccs[pl.ds(j * 16, 16)] = pos
            nccs[pl.ds(j * 16, 16)] = -pos

        def zero_acc(start, total):
            done = 0
            while done < total:
                step = min(RB * CHUNK, total - done)
                pltpu.sync_copy(rbuf.at[pl.ds(0, step)],
                                acc.at[pl.ds(start + done, step)])
                done += step

        @pl.when(sid < NS - 1)
        def _():
            zero_acc(sid * WB, WB)

        @pl.when(sid == NS - 1)
        def _():
            zero_acc((NS - 1) * WB, WB_LAST)

        pltpu.sync_copy(zcnt, cacc.at[pl.ds(sid * CNT_PER_TILE, CNT_PER_TILE)])

        plsc.subcore_barrier()

        def start_gather(kk, rb):
            base = row0 + kk * CHUNK
            pltpu.async_copy(node_h.at[pl.ds(base, CHUNK)],
                             rbuf.at[pl.ds(rb * CHUNK, CHUNK)], gsem.at[rb])
            pltpu.async_copy(lab_h.at[pl.ds(base, CHUNK)],
                             lbuf.at[rb, pl.ds(8, CHUNK)], gsem.at[rb])

        def wait_gather(rb):
            pltpu.make_async_copy(node_h.at[pl.ds(0, CHUNK)],
                                  rbuf.at[pl.ds(rb * CHUNK, CHUNK)],
                                  gsem.at[rb]).wait()
            pltpu.make_async_copy(lab_h.at[pl.ds(0, CHUNK)],
                                  lbuf.at[rb, pl.ds(8, CHUNK)],
                                  gsem.at[rb]).wait()

        def sum_dst(idx_ref):
            return acc.at[plsc.Indices(idx_ref, ignored_value=-1)]

        def cnt_dst(idx_ref):
            return cacc.at[plsc.Indices(idx_ref, ignored_value=-1)]

        def start_scatter(p):
            pltpu.async_copy(pbuf.at[p], sum_dst(mlab.at[p]), ssem.at[p],
                             add=True)
            pltpu.async_copy(ccs, cnt_dst(mlab.at[p]), ssem.at[p], add=True)
            pltpu.async_copy(nccs, cnt_dst(nlab.at[p]), ssem.at[p], add=True)

        def wait_scatter(p):
            pltpu.make_async_copy(pbuf.at[p], sum_dst(mlab.at[p]),
                                  ssem.at[p]).wait()
            pltpu.make_async_copy(ccs, cnt_dst(mlab.at[p]),
                                  ssem.at[p]).wait()
            pltpu.make_async_copy(nccs, cnt_dst(nlab.at[p]),
                                  ssem.at[p]).wait()

        def prologue(rb, p):
            for j in range(CHUNK // 16):
                cur = lbuf[rb, pl.ds(8 + 16 * j, 16)]
                prv = lbuf[rb, pl.ds(7 + 16 * j, 16)]
                succ = lbuf[rb, pl.ds(9 + 16 * j, 16)]
                eq = cur == prv
                if j == 0:
                    eq = jnp.logical_and(eq, lane0)
                sbuf[pl.ds(16 * j, 16)] = jnp.where(eq, 1.0, 0.0)
                ne = cur != succ
                if j == CHUNK // 16 - 1:
                    ne = jnp.logical_or(ne, lane15)
                mlab[p, pl.ds(16 * j, 16)] = jnp.where(ne, cur, neg1)
                nl = jnp.where(ne, succ, neg1)
                if j == CHUNK // 16 - 1:
                    nl = jnp.where(lane15, neg1, nl)
                nlab[p, pl.ds(16 * j, 16)] = nl

        def row_loop(rb, p, nrows):
            def row(i, acc8):
                mf = plsc.load_gather(sbuf, [jnp.full((16,), i, jnp.int32)])
                out = []
                for g in range(NG):
                    r = rbuf[rb * CHUNK + i, pl.ds(g * 16, 16)]
                    a = acc8[g] * mf + r
                    pbuf[p, i, pl.ds(g * 16, 16)] = a
                    out.append(a)
                return tuple(out)

            init = tuple(jnp.zeros((16,), jnp.float32) for _ in range(NG))
            lax.fori_loop(0, nrows, row, init)

        start_gather(0, 0)
        start_gather(1, 1)

        def ring_body(it, carry):
            for b in range(6):
                kk = it * 6 + b
                rb = b % RB
                p = b % PB

                @pl.when((kk >= 2) & (kk < NCH + 2))
                def _():
                    wait_scatter(p)

                @pl.when(kk + 2 < NCH)
                def _():
                    start_gather(kk + 2, (b + 2) % RB)

                @pl.when(kk < NCH)
                def _():
                    wait_gather(rb)
                    prologue(rb, p)
                    row_loop(rb, p, CHUNK)
                    start_scatter(p)
            return carry

        lax.fori_loop(0, (NCH + 2 + 5) // 6, ring_body, 0)

        if TAIL:
            tbase = row0 + NCH * CHUNK
            pltpu.sync_copy(node_h.at[pl.ds(tbase, TAIL)],
                            rbuf.at[pl.ds(0, TAIL)])
            pltpu.sync_copy(lab_h.at[pl.ds(tbase, TAIL)],
                            tlab.at[pl.ds(8, TAIL)])
            cur = tlab[pl.ds(8, 16)]
            prv = tlab[pl.ds(7, 16)]
            succ = tlab[pl.ds(9, 16)]
            eq = jnp.logical_and(cur == prv, lane0)
            sbuf[pl.ds(0, 16)] = jnp.where(eq, 1.0, 0.0)
            ne = jnp.logical_or(cur != succ, lane15)
            mtail[...] = jnp.where(ne, cur, neg1)
            ntail[...] = jnp.where(lane15, neg1, jnp.where(ne, succ, neg1))
            row_loop(0, 0, TAIL)
            pltpu.sync_copy(pbuf.at[0, pl.ds(0, TAIL)],
                            acc.at[plsc.Indices(mtail, ignored_value=-1)],
                            add=True)
            pltpu.sync_copy(ccs.at[pl.ds(0, TAIL)],
                            cacc.at[plsc.Indices(mtail, ignored_value=-1)],
                            add=True)
            pltpu.sync_copy(nccs.at[pl.ds(0, TAIL)],
                            cacc.at[plsc.Indices(ntail, ignored_value=-1)],
                            add=True)

        plsc.subcore_barrier()

        @pl.when(sid < NS - 1)
        def _():
            pltpu.sync_copy(acc.at[pl.ds(sid * WB, WB)],
                            sums_h.at[cid, pl.ds(sid * WB, WB)])

        @pl.when(sid == NS - 1)
        def _():
            pltpu.sync_copy(acc.at[pl.ds((NS - 1) * WB, WB_LAST)],
                            sums_h.at[cid, pl.ds((NS - 1) * WB, WB_LAST)])

        pltpu.sync_copy(cacc.at[pl.ds(sid * CNT_PER_TILE, CNT_PER_TILE)],
                        cnt_h.at[cid, pl.ds(sid * CNT_PER_TILE, CNT_PER_TILE)])

    return k(node_input, labels)


def _combine_body(sums_ref, cnt_ref, out_ref):
    s = sums_ref[0] + sums_ref[1]
    c = jnp.maximum(cnt_ref[0, :NSEG] + cnt_ref[1, :NSEG], 1.0)
    out_ref[...] = s / c[:, None]


def kernel(node_input, labels):
    labels = labels.astype(jnp.int32)
    sums, cnts = _phase_a(node_input, labels)
    out = pl.pallas_call(
        _combine_body,
        out_shape=jax.ShapeDtypeStruct((NSEG, DIM), jnp.float32),
    )(sums, cnts)
    return out
